# Initial kernel scaffold; baseline (speedup 1.0000x reference)
import jax
import jax.numpy as jnp
from jax.experimental import pallas as pl


def kernel(layers, W_conv, b_conv, W_cls, b_cls, W_reg, b_reg):
    raise NotImplementedError("write your pallas kernel here")


# speedup vs baseline: 11.9081x; 11.9081x over previous
import jax
import jax.numpy as jnp
import numpy as np
from jax.experimental import pallas as pl
from jax.experimental.pallas import tpu as pltpu

_STRIDE = 8
_SIZES = (32.0, 64.0, 128.0)
_RATIOS = (0.5, 1.0, 2.0)
_H = _W = 128
_PRE = 2000
_POST = 300
_NMS_T = 0.7
_MIN_AREA = 25.0
_NEG = -jnp.inf


def _anchor_tables():
    ys = (np.arange(_H) + 0.5) * _STRIDE
    xs = (np.arange(_W) + 0.5) * _STRIDE
    cx, cy = np.meshgrid(xs, ys)
    per = []
    for s in _SIZES:
        for r in _RATIOS:
            aw = s * np.sqrt(1.0 / r)
            ah = s * np.sqrt(r)
            per.append(np.stack([cx - aw / 2.0, cy - ah / 2.0,
                                 cx + aw / 2.0, cy + ah / 2.0], axis=-1))
    a = np.stack(per, axis=2).reshape(-1, 4).astype(np.float32)
    aw = a[:, 2] - a[:, 0]
    ah = a[:, 3] - a[:, 1]
    acx = a[:, 0] + np.float32(0.5) * aw
    acy = a[:, 1] + np.float32(0.5) * ah

    def plane(v):
        return np.ascontiguousarray(v.reshape(_H * _W, 9).T)

    return tuple(plane(v) for v in (aw, ah, acx, acy))


_AW9, _AH9, _ACX9, _ACY9 = _anchor_tables()


def _k1_body(xpad_ref, wmat_ref, bconv_ref, whead_ref, bhead_ref,
             aw_ref, ah_ref, acx_ref, acy_ref,
             key_ref, x1_ref, y1_ref, x2_ref, y2_ref):
    i = pl.program_id(0)
    h0 = pl.multiple_of(i * 8, 8)
    win = xpad_ref[:, pl.ds(h0, 16), :]
    slabs = []
    for dh in range(3):
        for dw in range(3):
            s = win[:, dh:dh + 8, dw:dw + 128]
            slabs.append(s.reshape(128, 1024))
    b = jnp.concatenate(slabs, axis=0)
    t = jax.lax.dot_general(wmat_ref[...], b, (((1,), (0,)), ((), ())),
                            preferred_element_type=jnp.float32)
    t = jnp.maximum(t + bconv_ref[...], 0.0)
    h45 = jax.lax.dot_general(whead_ref[...], t, (((1,), (0,)), ((), ())),
                              preferred_element_type=jnp.float32) + bhead_ref[...]
    for a in range(9):
        raw = h45[a:a + 1, :]
        dx = h45[9 + 4 * a:10 + 4 * a, :]
        dy = h45[10 + 4 * a:11 + 4 * a, :]
        dwc = jnp.clip(h45[11 + 4 * a:12 + 4 * a, :], -4.0, 4.0)
        dhc = jnp.clip(h45[12 + 4 * a:13 + 4 * a, :], -4.0, 4.0)
        aw = aw_ref[a:a + 1, :]
        ah = ah_ref[a:a + 1, :]
        acx = acx_ref[a:a + 1, :]
        acy = acy_ref[a:a + 1, :]
        cx = dx * aw + acx
        cy = dy * ah + acy
        w = jnp.exp(dwc) * aw
        h = jnp.exp(dhc) * ah
        x1 = jnp.clip(cx - 0.5 * w, 0.0, 1024.0)
        y1 = jnp.clip(cy - 0.5 * h, 0.0, 1024.0)
        x2 = jnp.clip(cx + 0.5 * w, 0.0, 1024.0)
        y2 = jnp.clip(cy + 0.5 * h, 0.0, 1024.0)
        area = (x2 - x1) * (y2 - y1)
        key_ref[a:a + 1, :] = jnp.where(area >= _MIN_AREA, raw, _NEG)
        x1_ref[a:a + 1, :] = x1
        y1_ref[a:a + 1, :] = y1
        x2_ref[a:a + 1, :] = x2
        y2_ref[a:a + 1, :] = y2


def _rev_lanes(x):
    n = x.shape[-1]
    lane = jax.lax.broadcasted_iota(jnp.int32, x.shape, x.ndim - 1)
    d = 1
    while d < n:
        x = jnp.where((lane & d) == 0, jnp.roll(x, -d, axis=-1),
                      jnp.roll(x, d, axis=-1))
        d *= 2
    return x


def _cmpex(key, idx, payloads, d, k):
    lane = jax.lax.broadcasted_iota(jnp.int32, key.shape, key.ndim - 1)
    low = (lane & d) == 0

    def partner(x):
        return jnp.where(low, jnp.roll(x, -d, axis=-1), jnp.roll(x, d, axis=-1))

    kp = partner(key)
    ip = partner(idx)
    pp = [partner(p) for p in payloads]
    self_better = (key > kp) | ((key == kp) & (idx < ip))
    keeps_max = ((lane & k) == 0) == low
    take = keeps_max != self_better
    key = jnp.where(take, kp, key)
    idx = jnp.where(take, ip, idx)
    payloads = [jnp.where(take, p, s) for s, p in zip(payloads, pp)]
    return key, idx, payloads


def _sort_rows_desc(key, idx, payloads):
    n = key.shape[-1]
    k = 2
    while k <= n:
        d = k // 2
        while d >= 1:
            key, idx, payloads = _cmpex(key, idx, payloads, d, k)
            d //= 2
        k *= 2
    return key, idx, payloads


def _merge_desc(key, idx, payloads):
    n = key.shape[-1]
    d = n // 2
    while d >= 1:
        key, idx, payloads = _cmpex(key, idx, payloads, d, n)
        d //= 2
    return key, idx, payloads


def _k2_body(key_ref, x1_ref, y1_ref, x2_ref, y2_ref,
             okey_ref, ox1_ref, oy1_ref, ox2_ref, oy2_ref):
    key = key_ref[...]
    ridx = jax.lax.broadcasted_iota(jnp.int32, key.shape, 0)
    cidx = jax.lax.broadcasted_iota(jnp.int32, key.shape, 1)
    idx = ridx * 1024 + cidx
    pay = [x1_ref[...], y1_ref[...], x2_ref[...], y2_ref[...]]
    key, idx, pay = _sort_rows_desc(key, idx, pay)
    while key.shape[0] > 1:
        nl = key.shape[0]
        half = nl // 2

        def fold(x):
            return jnp.concatenate([x[:half], _rev_lanes(x[half:2 * half])],
                                   axis=-1)

        mk, mi, mp = _merge_desc(fold(key), fold(idx), [fold(p) for p in pay])
        if mk.shape[-1] > 2048:
            mk = mk[:, :2048]
            mi = mi[:, :2048]
            mp = [p[:, :2048] for p in mp]
        if nl > 2 * half:
            rest = slice(2 * half, nl)
            mk = jnp.concatenate([mk, key[rest]], axis=0)
            mi = jnp.concatenate([mi, idx[rest]], axis=0)
            mp = [jnp.concatenate([p, q[rest]], axis=0) for p, q in zip(mp, pay)]
        key, idx, pay = mk, mi, mp
    okey_ref[...] = key
    ox1_ref[...] = pay[0]
    oy1_ref[...] = pay[1]
    ox2_ref[...] = pay[2]
    oy2_ref[...] = pay[3]


def _k3_body(key_ref, x1r_ref, y1r_ref, x2r_ref, y2r_ref,
             x1c_ref, y1c_ref, x2c_ref, y2c_ref,
             oscore_ref, ox1_ref, oy1_ref, ox2_ref, oy2_ref,
             m_ref):
    x1r = x1r_ref[...]
    y1r = y1r_ref[...]
    x2r = x2r_ref[...]
    y2r = y2r_ref[...]
    area_r = (x2r - x1r) * (y2r - y1r)
    col = jax.lax.broadcasted_iota(jnp.int32, (128, 2048), 1)
    for bk in range(16):
        r0 = bk * 128
        x1c = x1c_ref[r0:r0 + 128, :]
        y1c = y1c_ref[r0:r0 + 128, :]
        x2c = x2c_ref[r0:r0 + 128, :]
        y2c = y2c_ref[r0:r0 + 128, :]
        area_c = (x2c - x1c) * (y2c - y1c)
        ix1 = jnp.maximum(x1c, x1r)
        iy1 = jnp.maximum(y1c, y1r)
        ix2 = jnp.minimum(x2c, x2r)
        iy2 = jnp.minimum(y2c, y2r)
        iw = jnp.maximum(ix2 - ix1, 0.0)
        ih = jnp.maximum(iy2 - iy1, 0.0)
        inter = iw * ih
        union = area_c + area_r - inter
        iou = inter / jnp.maximum(union, 1e-9)
        row = jax.lax.broadcasted_iota(jnp.int32, (128, 2048), 0) + r0
        mblk = jnp.where((iou > _NMS_T) & (col > row), 1.0, 0.0)
        m_ref[r0:r0 + 128] = mblk.reshape(128, 16, 128)

    key = key_ref[...]
    lane = jax.lax.broadcasted_iota(jnp.int32, (1, 2048), 1)
    jgrid = jax.lax.broadcasted_iota(jnp.int32, (16, 128), 0) * 128 + \
        jax.lax.broadcasted_iota(jnp.int32, (16, 128), 1)
    valid = (key.reshape(16, 128) > _NEG) & (jgrid < _PRE)
    keep0 = jnp.where(valid, 1.0, 0.0)

    def body(i, keep):
        rowm = m_ref[i]
        ki = jnp.sum(jnp.where(jgrid == i, keep, 0.0))
        return keep * (1.0 - ki * rowm)

    keep = jax.lax.fori_loop(0, _PRE, body, keep0)
    fkey = jnp.where(keep.reshape(1, 2048) > 0.0, key, _NEG)
    skey, _, spay = _sort_rows_desc(fkey, lane, [x1r, y1r, x2r, y2r])
    oscore_ref[...] = jnp.where(skey[:, :512] == _NEG, _NEG,
                                jax.nn.sigmoid(skey[:, :512]))
    ox1_ref[...] = spay[0][:, :512]
    oy1_ref[...] = spay[1][:, :512]
    ox2_ref[...] = spay[2][:, :512]
    oy2_ref[...] = spay[3][:, :512]


def kernel(layers, W_conv, b_conv, W_cls, b_cls, W_reg, b_reg):
    f32 = jnp.float32
    x = layers[0]
    xpad = jnp.pad(x, ((0, 0), (1, 7), (1, 1)))
    wmat = jnp.transpose(W_conv, (2, 3, 1, 0)).reshape(1152, 128).T
    whead = jnp.concatenate([W_cls.reshape(9, 128), W_reg.reshape(36, 128)], axis=0)
    bhead = jnp.concatenate([b_cls, b_reg]).reshape(45, 1)
    bconv = b_conv.reshape(128, 1)
    aw9 = jnp.asarray(_AW9)
    ah9 = jnp.asarray(_AH9)
    acx9 = jnp.asarray(_ACX9)
    acy9 = jnp.asarray(_ACY9)

    full = lambda s: pl.BlockSpec(s, lambda i: (0,) * len(s))
    tbl = pl.BlockSpec((9, 1024), lambda i: (0, i))
    k1 = pl.pallas_call(
        _k1_body,
        grid=(16,),
        in_specs=[full((128, 136, 130)), full((128, 1152)), full((128, 1)),
                  full((45, 128)), full((45, 1)), tbl, tbl, tbl, tbl],
        out_specs=[tbl] * 5,
        out_shape=[jax.ShapeDtypeStruct((9, 16384), f32)] * 5,
    )
    key9, x19, y19, x29, y29 = k1(xpad, wmat, bconv, whead, bhead,
                                  aw9, ah9, acx9, acy9)

    r = lambda v: v.reshape(144, 1024)
    k2 = pl.pallas_call(
        _k2_body,
        out_shape=[jax.ShapeDtypeStruct((1, 2048), f32)] * 5,
    )
    key2, x12, y12, x22, y22 = k2(r(key9), r(x19), r(y19), r(x29), r(y29))

    k3 = pl.pallas_call(
        _k3_body,
        out_shape=[jax.ShapeDtypeStruct((1, 512), f32)] * 5,
        scratch_shapes=[pltpu.VMEM((2048, 16, 128), f32)],
    )
    tcol = lambda v: v.reshape(2048, 1)
    osc, ox1, oy1, ox2, oy2 = k3(key2, x12, y12, x22, y22,
                                 tcol(x12), tcol(y12), tcol(x22), tcol(y22))

    props = jnp.stack([ox1[0, :_POST], oy1[0, :_POST],
                       ox2[0, :_POST], oy2[0, :_POST]], axis=-1)
    return props[None], osc[:, :_POST]
